# Initial kernel scaffold; baseline (speedup 1.0000x reference)
import jax
import jax.numpy as jnp
from jax.experimental import pallas as pl


def kernel(x, head_conn, head_mem, proj_conn, proj_mem, agg_conn, agg_mem):
    raise NotImplementedError("write your pallas kernel here")


# speedup vs baseline: 439.0541x; 439.0541x over previous
import jax
import jax.numpy as jnp
from jax import lax
from jax.experimental import pallas as pl
from jax.experimental.pallas import tpu as pltpu
from jax.experimental.pallas import tpu_sc as plsc

S = 512
B_IN = 128
H = 16
NPB = 9
HEAD_NB = 12
PROJ_NB = 12
WB = 5
AGG_NB = 12
NW = 32
RPW = S // NW


def _body(aq_h, ak_h, ar_h, xp_h, pcw_h, pcs_h, ac_h, hm_h, proj_h, agg_h,
          out_h,
          aq_v, ak_v, ar_v, xp_v, pcw_v, pcs_v, ac_v, hm_v,
          lane_v, idx_v, rows_v, aggin_v, orow_v, sem):
    pltpu.sync_copy(aq_h, aq_v)
    pltpu.sync_copy(ak_h, ak_v)
    pltpu.sync_copy(ar_h, ar_v)
    pltpu.sync_copy(xp_h, xp_v)
    pltpu.sync_copy(pcw_h, pcw_v)
    pltpu.sync_copy(pcs_h, pcs_v)
    pltpu.sync_copy(ac_h, ac_v)
    pltpu.sync_copy(hm_h, hm_v)

    wid = lax.axis_index("s") * 2 + lax.axis_index("c")
    iot = lax.iota(jnp.int32, 16)
    hoff = iot * 4096

    def row_fn(i, _):
        q = wid + NW * i
        base = hoff + aq_v[pl.ds(pl.multiple_of(q * 16, 8), 16)]

        def kbody(k, c):
            mv, js = c
            idx = (base + ak_v[pl.ds(pl.multiple_of(k * 16, 8), 16)]
                   + ar_v[pl.ds(pl.multiple_of((q - k) * 16, 8), 16)])
            s = jnp.sum(plsc.load_gather(hm_v, [idx]))
            upd = s > mv
            return (jnp.where(upd, s, mv), jnp.where(upd, k, js))

        mv, js = lax.fori_loop(0, q + 1, kbody,
                               (jnp.float32(-1.0), jnp.int32(0)))

        t = mv.astype(jnp.int32)
        r = mv - t.astype(jnp.float32)
        up = (r > 0.5) | ((r == 0.5) & ((t & 1) == 1))
        vc = jnp.clip(t + up.astype(jnp.int32), 0, H)

        def pblk(b, _c):
            b0 = b * 16
            acc = jnp.zeros((16,), jnp.int32)
            for tp in range(PROJ_NB):
                off = pl.ds(pl.multiple_of(tp * 128 + b0, 8), 16)
                wv = plsc.load_gather(xp_v, [js * 4 + pcw_v[off]])
                acc = acc + (((wv >> pcs_v[off]) & 1) << (PROJ_NB - 1 - tp))
            bo = pl.ds(pl.multiple_of(b0, 8), 16)
            idx_v[bo] = (b0 + iot) * 32 + (acc >> 7)
            lane_v[bo] = acc & 127
            return 0

        lax.fori_loop(0, 8, pblk, 0)
        pltpu.async_copy(proj_h.at[idx_v], rows_v, sem).wait()

        mvpos = mv > 0.0

        def vblk(b, _c):
            b0 = b * 16
            bo = pl.ds(pl.multiple_of(b0, 8), 16)
            vals = plsc.load_gather(rows_v, [b0 + iot, lane_v[bo]])
            aggin_v[bo] = jnp.where((vals > 0.5) & mvpos, 1, 0)
            return 0

        lax.fori_loop(0, 8, vblk, 0)
        sh = jnp.maximum(4 - iot, 0)
        aggin_v[pl.ds(128, 16)] = jnp.where(iot < WB, (vc >> sh) & 1, 0)

        def ablk(b, _c):
            b0 = b * 16
            acc = jnp.zeros((16,), jnp.int32)
            for tp in range(AGG_NB):
                off = pl.ds(pl.multiple_of(tp * 128 + b0, 8), 16)
                acc = acc + (plsc.load_gather(aggin_v, [ac_v[off]])
                             << (AGG_NB - 1 - tp))
            bo = pl.ds(pl.multiple_of(b0, 8), 16)
            idx_v[bo] = (b0 + iot) * 32 + (acc >> 7)
            lane_v[bo] = acc & 127
            return 0

        lax.fori_loop(0, 8, ablk, 0)
        pltpu.async_copy(agg_h.at[idx_v], rows_v, sem).wait()

        def oblk(b, _c):
            b0 = b * 16
            bo = pl.ds(pl.multiple_of(b0, 8), 16)
            orow_v[bo] = plsc.load_gather(rows_v, [b0 + iot, lane_v[bo]])
            return 0

        lax.fori_loop(0, 8, oblk, 0)
        pltpu.sync_copy(orow_v, out_h.at[q])
        return 0

    lax.fori_loop(0, RPW, row_fn, 0)


def kernel(x, head_conn, head_mem, proj_conn, proj_mem, agg_conn, agg_mem):
    pw = (2 ** jnp.arange(HEAD_NB - 1, -1, -1)).astype(jnp.int32)
    mq = (head_conn < B_IN).astype(jnp.int32)
    mk = ((head_conn >= B_IN) & (head_conn < 2 * B_IN)).astype(jnp.int32)
    mr = (head_conn >= 2 * B_IN).astype(jnp.int32)
    iq = jnp.clip(head_conn, 0, B_IN - 1)
    ik = jnp.clip(head_conn - B_IN, 0, B_IN - 1)
    ir = jnp.clip(head_conn - 2 * B_IN, 0, NPB - 1)
    addr_q = jnp.sum(x[:, iq] * (pw * mq)[None], axis=-1).astype(jnp.int32)
    addr_k = jnp.sum(x[:, ik] * (pw * mk)[None], axis=-1).astype(jnp.int32)
    d = jnp.arange(S)
    relb = ((d[:, None] >> jnp.arange(NPB - 1, -1, -1)[None]) & 1).astype(jnp.int32)
    addr_r = jnp.sum(relb[:, ir] * (pw * mr)[None], axis=-1).astype(jnp.int32)

    xp = jnp.sum(x.reshape(S, 4, 32)
                 * (jnp.int32(1) << jnp.arange(32, dtype=jnp.int32)),
                 axis=-1, dtype=jnp.int32).reshape(-1)
    pcw = (proj_conn >> 5).T.reshape(-1).astype(jnp.int32)
    pcs = (proj_conn & 31).T.reshape(-1).astype(jnp.int32)
    ac = agg_conn.T.reshape(-1).astype(jnp.int32)
    hm = head_mem.reshape(-1)
    proj2 = proj_mem.reshape(-1).reshape(B_IN * 32, 128)
    agg2 = agg_mem.reshape(-1).reshape(B_IN * 32, 128)

    mesh = plsc.VectorSubcoreMesh(core_axis_name="c", subcore_axis_name="s")
    f = pl.kernel(
        _body,
        out_type=jax.ShapeDtypeStruct((S, B_IN), jnp.float32),
        mesh=mesh,
        compiler_params=pltpu.CompilerParams(needs_layout_passes=False),
        scratch_types=[
            pltpu.VMEM((S * H,), jnp.int32),
            pltpu.VMEM((S * H,), jnp.int32),
            pltpu.VMEM((S * H,), jnp.int32),
            pltpu.VMEM((S * 4,), jnp.int32),
            pltpu.VMEM((PROJ_NB * B_IN,), jnp.int32),
            pltpu.VMEM((PROJ_NB * B_IN,), jnp.int32),
            pltpu.VMEM((AGG_NB * B_IN,), jnp.int32),
            pltpu.VMEM((H * 4096,), jnp.float32),
            pltpu.VMEM((B_IN,), jnp.int32),
            pltpu.VMEM((B_IN,), jnp.int32),
            pltpu.VMEM((B_IN, 128), jnp.float32),
            pltpu.VMEM((B_IN + 16,), jnp.int32),
            pltpu.VMEM((B_IN,), jnp.float32),
            pltpu.SemaphoreType.DMA,
        ],
    )
    return f(addr_q.reshape(-1), addr_k.reshape(-1), addr_r.reshape(-1),
             xp, pcw, pcs, ac, hm, proj2, agg2)
